# baseline (speedup 1.0000x reference)
import jax
import jax.numpy as jnp
from jax.experimental import pallas as pl


def _bias_kernel(h_ref, b_ref, o_ref):
    o_ref[...] = h_ref[...] + b_ref[...]


def _graph_conv(src, dst, num_src, num_dst, feat_src, W, b):
    E = src.shape[0]
    ones = jnp.ones((E,), dtype=feat_src.dtype)
    deg_out = jax.ops.segment_sum(ones, src, num_segments=num_src)
    norm_src = jnp.power(jnp.clip(deg_out, 1.0, None), -0.5)
    h = feat_src * norm_src[:, None]
    in_f, out_f = W.shape
    if in_f > out_f:
        h = h @ W
        rst = jax.ops.segment_sum(jnp.take(h, src, axis=0), dst, num_segments=num_dst)
    else:
        rst = jax.ops.segment_sum(jnp.take(h, src, axis=0), dst, num_segments=num_dst)
        rst = rst @ W
    deg_in = jax.ops.segment_sum(ones, dst, num_segments=num_dst)
    norm_dst = jnp.power(jnp.clip(deg_in, 1.0, None), -0.5)
    rst = rst * norm_dst[:, None]
    return rst, b


def kernel(x, mfg0_src, mfg0_dst, mfg0_num_dst, mfg1_src, mfg1_dst, mfg1_num_dst, W1, b1, W2, b2):
    NUM_DST0 = 20000
    NUM_DST1 = 4096
    h, bb = _graph_conv(mfg0_src, mfg0_dst, x.shape[0], NUM_DST0, x, W1, b1)
    h = jax.nn.relu(h + bb)
    h, bb = _graph_conv(mfg1_src, mfg1_dst, NUM_DST0, NUM_DST1, h, W2, b2)
    out = pl.pallas_call(
        _bias_kernel,
        out_shape=jax.ShapeDtypeStruct(h.shape, h.dtype),
    )(h, jnp.broadcast_to(bb, h.shape))
    return out


# speedup vs baseline: 3.1818x; 3.1818x over previous
import functools

import jax
import jax.numpy as jnp
from jax import lax
from jax.experimental import pallas as pl
from jax.experimental.pallas import tpu as pltpu
from jax.experimental.pallas import tpu_sc as plsc

_NC, _NS, _L = 2, 16, 16
_NW = _NC * _NS

_E0 = 320000
_N0 = 100000
_E1 = 65536
_N1 = 20000
_ND0 = 20000
_ND1 = 4096
_K = 128


def _hist_phase(w, src_hbm, out_hbm, chunk_v, hist_v, E, B, chunk_len, nchunks):
    ones = jnp.ones((_L,), jnp.float32)
    base = w * (E // _NW)

    def zero_hist(i, _):
        hist_v[pl.ds(i * _L, _L)] = jnp.zeros((_L,), jnp.float32)
        return 0
    lax.fori_loop(0, B // _L, zero_hist, 0)

    for k in range(nchunks):
        pltpu.sync_copy(src_hbm.at[pl.ds(base + k * chunk_len, chunk_len)],
                        chunk_v.at[pl.ds(0, chunk_len)])

        def scan(i, _):
            idx = chunk_v[pl.ds(i * _L, _L)]
            plsc.addupdate_scatter(hist_v, [idx], ones)
            return 0
        lax.fori_loop(0, chunk_len // _L, scan, 0)

    pltpu.sync_copy(hist_v.at[pl.ds(0, B)], out_hbm.at[pl.ds(w * B, B)])


def _hist_body(src0_hbm, src1_hbm, d0_hbm, d1_hbm, chunk_v, hist_v):
    c = lax.axis_index("c")
    s = lax.axis_index("s")
    w = s * _NC + c
    _hist_phase(w, src0_hbm, d0_hbm, chunk_v, hist_v, _E0, _N0, 2000, 5)
    _hist_phase(w, src1_hbm, d1_hbm, chunk_v, hist_v, _E1, _N1, 2048, 1)


_hist_call = functools.partial(
    pl.kernel,
    _hist_body,
    out_type=[
        jax.ShapeDtypeStruct((_NW * _N0,), jnp.float32),
        jax.ShapeDtypeStruct((_NW * _N1,), jnp.float32),
    ],
    mesh=plsc.VectorSubcoreMesh(core_axis_name="c", subcore_axis_name="s"),
    scratch_types=[
        pltpu.VMEM((2048,), jnp.int32),
        pltpu.VMEM((_N0,), jnp.float32),
    ],
    compiler_params=pltpu.CompilerParams(needs_layout_passes=False),
    name="sc_degree_hists",
)


def _make_spmm(E, n_table, per_core, chunk, nchunks, name):
    e_per_tile = E // _NS
    assert e_per_tile == chunk * nchunks
    dump = per_core
    acc_rows = ((per_core + 16 + 127) // 128) * 128
    stripe = acc_rows // _NS
    out_stripe = (per_core // _NS) // 8 * 8
    out_rem = per_core - out_stripe * _NS
    nb_max = chunk // _K

    def body(src_hbm, dst_hbm, table_hbm, out_hbm, hist_hbm,
             srcv, dstv, csrc, cdlf, cdl2, rows, histv, acc_sh, sem):
        c = lax.axis_index("c")
        s = lax.axis_index("s")
        w = s * _NC + c
        lo = c * per_core
        zero16 = jnp.zeros((_L,), jnp.float32)
        ones16 = jnp.ones((_L,), jnp.float32)

        def zrow(r, _):
            for u in range(8):
                rows[r, pl.ds(u * _L, _L)] = zero16
            return 0
        lax.fori_loop(0, _K, zrow, 0)
        for off in range(0, stripe, _K):
            n = min(_K, stripe - off)
            pltpu.sync_copy(rows.at[pl.ds(0, n), :],
                            acc_sh.at[pl.ds(s * stripe + off, n), :])

        def zhist(i, _):
            histv[pl.ds(i * _L, _L)] = zero16
            return 0
        lax.fori_loop(0, (per_core + 16) // _L, zhist, 0)
        plsc.subcore_barrier()

        for k in range(nchunks):
            base_e = s * e_per_tile + k * chunk
            pltpu.sync_copy(src_hbm.at[pl.ds(base_e, chunk)],
                            srcv.at[pl.ds(0, chunk)])
            pltpu.sync_copy(dst_hbm.at[pl.ds(base_e, chunk)],
                            dstv.at[pl.ds(0, chunk)])

            def pre(i, _):
                csrc[pl.ds(i * _L, _L)] = jnp.zeros((_L,), jnp.int32)
                cdlf[pl.ds(i * _L, _L)] = jnp.full((_L,), dump, jnp.int32)
                return 0
            lax.fori_loop(0, chunk // _L, pre, 0)

            def scan(i, cnt):
                d16 = dstv[pl.ds(i * _L, _L)]
                s16 = srcv[pl.ds(i * _L, _L)]
                m = (d16 >= lo) & (d16 < lo + per_core)
                dl = d16 - lo
                plsc.store_compressed(csrc.at[pl.ds(cnt, _L)], s16, mask=m)
                plsc.store_compressed(cdlf.at[pl.ds(cnt, _L)], dl, mask=m)
                dl_h = jnp.where(m, dl, per_core)
                plsc.addupdate_scatter(histv, [dl_h], ones16, mask=m)
                return cnt + jnp.sum(m.astype(jnp.int32))
            cnt = lax.fori_loop(0, chunk // _L, scan, 0)

            def tocdl2(j, _):
                cdl2[j // 8, pl.ds((j % 8) * _L, _L)] = cdlf[pl.ds(j * _L, _L)]
                return 0
            lax.fori_loop(0, chunk // _L, tocdl2, 0)

            nb = (cnt + _K - 1) // _K

            def batch(b, _):
                pltpu.async_copy(table_hbm.at[csrc.at[pl.ds(b * _K, _K)]],
                                 rows, sem).wait()
                pltpu.sync_copy(rows, acc_sh.at[cdl2.at[b]], add=True)
                return 0
            lax.fori_loop(0, nb, batch, 0)

        plsc.subcore_barrier()

        for off in range(0, out_stripe, _K):
            n = min(_K, out_stripe - off)
            pltpu.sync_copy(acc_sh.at[pl.ds(s * out_stripe + off, n), :],
                            out_hbm.at[pl.ds(c * per_core + s * out_stripe + off, n), :])
        if out_rem:
            @pl.when(s == 0)
            def _copy_rem():
                pltpu.sync_copy(
                    acc_sh.at[pl.ds(out_stripe * _NS, out_rem), :],
                    out_hbm.at[pl.ds(c * per_core + out_stripe * _NS, out_rem), :])
        pltpu.sync_copy(histv.at[pl.ds(0, per_core)],
                        hist_hbm.at[pl.ds(w * per_core, per_core)])

    return functools.partial(
        pl.kernel,
        body,
        out_type=[
            jax.ShapeDtypeStruct((_NC * per_core, 128), jnp.float32),
            jax.ShapeDtypeStruct((_NW * per_core,), jnp.float32),
        ],
        mesh=plsc.VectorSubcoreMesh(core_axis_name="c", subcore_axis_name="s"),
        scratch_types=[
            pltpu.VMEM((chunk,), jnp.int32),
            pltpu.VMEM((chunk,), jnp.int32),
            pltpu.VMEM((chunk,), jnp.int32),
            pltpu.VMEM((chunk,), jnp.int32),
            pltpu.VMEM((nb_max, _K), jnp.int32),
            pltpu.VMEM((_K, 128), jnp.float32),
            pltpu.VMEM((per_core + 16,), jnp.float32),
            pltpu.VMEM_SHARED((acc_rows, 128), jnp.float32),
            pltpu.SemaphoreType.DMA,
        ],
        compiler_params=pltpu.CompilerParams(needs_layout_passes=False),
        name=name,
    )


_spmm0_call = _make_spmm(_E0, _N0, _ND0 // 2, 2000, 10, "sc_spmm0")
_spmm1_call = _make_spmm(_E1, _N1, _ND1 // 2, 2048, 2, "sc_spmm1")


def kernel(x, mfg0_src, mfg0_dst, mfg0_num_dst, mfg1_src, mfg1_dst, mfg1_num_dst, W1, b1, W2, b2):
    d0p, d1p = _hist_call()(mfg0_src, mfg1_src)
    deg_src0 = jnp.sum(d0p.reshape(_NW, _N0), axis=0)
    deg_src1 = jnp.sum(d1p.reshape(_NW, _N1), axis=0)

    norm_src0 = jax.lax.rsqrt(jnp.clip(deg_src0, 1.0, None))
    h = x * norm_src0[:, None]
    agg0, hd0 = _spmm0_call()(mfg0_src, mfg0_dst, h)
    r0 = hd0.reshape(_NS, _NC, _ND0 // 2)
    deg_in0 = jnp.concatenate([jnp.sum(r0[:, 0], axis=0), jnp.sum(r0[:, 1], axis=0)])
    rst = agg0 @ W1
    rst = rst * jax.lax.rsqrt(jnp.clip(deg_in0, 1.0, None))[:, None] + b1
    h = jax.nn.relu(rst)

    norm_src1 = jax.lax.rsqrt(jnp.clip(deg_src1, 1.0, None))
    h = h * norm_src1[:, None]
    h = h @ W2
    hpad = jnp.concatenate([h, jnp.zeros((_N1, 64), jnp.float32)], axis=1)
    agg1, hd1 = _spmm1_call()(mfg1_src, mfg1_dst, hpad)
    r1 = hd1.reshape(_NS, _NC, _ND1 // 2)
    deg_in1 = jnp.concatenate([jnp.sum(r1[:, 0], axis=0), jnp.sum(r1[:, 1], axis=0)])
    rst = agg1[:, :64] * jax.lax.rsqrt(jnp.clip(deg_in1, 1.0, None))[:, None] + b2
    return rst
